# Initial kernel scaffold; baseline (speedup 1.0000x reference)
import jax
import jax.numpy as jnp
from jax.experimental import pallas as pl


def kernel(x, edge_index, W1, b1, W2, b2):
    raise NotImplementedError("write your pallas kernel here")


# speedup vs baseline: 4.4593x; 4.4593x over previous
import functools

import jax
import jax.numpy as jnp
from jax import lax
from jax.experimental import pallas as pl
from jax.experimental.pallas import tpu as pltpu
from jax.experimental.pallas import tpu_sc as plsc

N_NODES = 10000
N_EDGES = 320000
NFEATS = 128
HIDDEN = 64
CHANNELS = 8

NC = 2
NS = 16
NW = NC * NS
EDGES_PER_W = N_EDGES // NW
CHUNK = 200
NCHUNK = EDGES_PER_W // CHUNK
ROWS_PER_TILE = 624
ROWS_LAST = N_NODES - 15 * ROWS_PER_TILE

_MESH = plsc.VectorSubcoreMesh(core_axis_name="c", subcore_axis_name="s",
                               num_cores=NC, num_subcores=NS)
_SC_PARAMS = pltpu.CompilerParams(needs_layout_passes=False)


def _k1_body(x_ref, w1_ref, t_ref):
    xv = x_ref[...]
    w = w1_ref[...]
    t_ref[:, :HIDDEN] = jnp.dot(xv, w[:NFEATS], preferred_element_type=jnp.float32)
    t_ref[:, HIDDEN:] = jnp.dot(xv, w[NFEATS:], preferred_element_type=jnp.float32)


def _node_table(x, W1):
    return pl.pallas_call(
        _k1_body,
        out_shape=jax.ShapeDtypeStruct((N_NODES, 2 * HIDDEN), jnp.float32),
    )(x, W1)


def _k2_body(t_hbm, row_hbm, col_hbm, h_hbm, idxr, idxc, bufr, bufc, gbuf, sem):
    w = lax.axis_index("c") * NS + lax.axis_index("s")
    for k in range(NCHUNK):
        base = pl.multiple_of(w * EDGES_PER_W + k * CHUNK, 8)
        sl = pl.ds(base, CHUNK)
        pltpu.sync_copy(row_hbm.at[sl], idxr)
        pltpu.sync_copy(col_hbm.at[sl], idxc)
        d1 = pltpu.async_copy(t_hbm.at[idxr], bufr, sem)
        d2 = pltpu.async_copy(t_hbm.at[idxc], bufc, sem)
        d1.wait()
        d2.wait()

        def body(e, carry):
            for j in range(HIDDEN // 16):
                a = bufr[e, pl.ds(j * 16, 16)]
                b = bufc[e, pl.ds(HIDDEN + j * 16, 16)]
                gbuf[e, pl.ds(j * 16, 16)] = a + b
            return carry

        lax.fori_loop(0, CHUNK, body, 0)
        pltpu.sync_copy(gbuf, h_hbm.at[sl])


_gather_sum = functools.partial(
    pl.kernel,
    compiler_params=_SC_PARAMS,
    out_type=jax.ShapeDtypeStruct((N_EDGES, HIDDEN), jnp.float32),
    mesh=_MESH,
    scratch_types=[
        pltpu.VMEM((CHUNK,), jnp.int32),
        pltpu.VMEM((CHUNK,), jnp.int32),
        pltpu.VMEM((CHUNK, 2 * HIDDEN), jnp.float32),
        pltpu.VMEM((CHUNK, 2 * HIDDEN), jnp.float32),
        pltpu.VMEM((CHUNK, HIDDEN), jnp.float32),
        pltpu.SemaphoreType.DMA,
    ],
)(_k2_body)


_K3_BE = 4000


def _k3_body(h_ref, b1_ref, w2_ref, b2_ref, o_ref):
    h = jnp.maximum(h_ref[...] + b1_ref[...][None, :], 0.0)
    s = jnp.dot(h, w2_ref[...], preferred_element_type=jnp.float32)
    o_ref[...] = jnp.exp(s + b2_ref[...][None, :])


def _edge_exp_scores(h, b1, W2, b2):
    grid = (N_EDGES // _K3_BE,)
    return pl.pallas_call(
        _k3_body,
        grid=grid,
        in_specs=[
            pl.BlockSpec((_K3_BE, HIDDEN), lambda i: (i, 0)),
            pl.BlockSpec((HIDDEN,), lambda i: (0,)),
            pl.BlockSpec((HIDDEN, CHANNELS), lambda i: (0, 0)),
            pl.BlockSpec((CHANNELS,), lambda i: (0,)),
        ],
        out_specs=pl.BlockSpec((_K3_BE, CHANNELS), lambda i: (i, 0)),
        out_shape=jax.ShapeDtypeStruct((N_EDGES, CHANNELS), jnp.float32),
    )(h, b1, W2, b2)


_NFLAT = N_NODES * CHANNELS
_CHUNK4 = 400
_NCHUNK4 = EDGES_PER_W // _CHUNK4
_GROUPS4 = _CHUNK4 * CHANNELS // 16
_ZGROUPS = _NFLAT // 16


def _k4_body(ef_hbm, row_hbm, sp_hbm, idx, ebuf, sbuf):
    w = lax.axis_index("c") * NS + lax.axis_index("s")
    iota = lax.iota(jnp.int32, 16)
    rsel = iota >> 3
    csel = iota & 7
    mlo = iota < 8
    mhi = iota >= 8
    zero16 = jnp.zeros((16,), jnp.float32)

    def zbody(i, carry):
        sbuf[pl.ds(pl.multiple_of(16 * i, 8), 16)] = zero16
        return carry

    lax.fori_loop(0, _ZGROUPS, zbody, 0)

    for k in range(_NCHUNK4):
        base = pl.multiple_of(w * EDGES_PER_W + k * _CHUNK4, 8)
        fsl = pl.ds(pl.multiple_of(base * CHANNELS, 8), _CHUNK4 * CHANNELS)
        pltpu.sync_copy(row_hbm.at[pl.ds(base, _CHUNK4)], idx)
        pltpu.sync_copy(ef_hbm.at[fsl], ebuf)

        def body(j, carry):
            off = pl.multiple_of(16 * j, 8)
            rows2 = plsc.load_gather(idx, [2 * j + rsel])
            eidx = rows2 * CHANNELS + csel
            ev = ebuf[pl.ds(off, 16)]
            plsc.addupdate_scatter(sbuf, [eidx], ev, mask=mlo)
            plsc.addupdate_scatter(sbuf, [eidx], ev, mask=mhi)
            return carry

        lax.fori_loop(0, _GROUPS4, body, 0)

    pltpu.sync_copy(sbuf, sp_hbm.at[w])


_segment_sums = functools.partial(
    pl.kernel,
    compiler_params=_SC_PARAMS,
    out_type=jax.ShapeDtypeStruct((NW, _NFLAT), jnp.float32),
    mesh=_MESH,
    scratch_types=[
        pltpu.VMEM((_CHUNK4,), jnp.int32),
        pltpu.VMEM((_CHUNK4 * CHANNELS,), jnp.float32),
        pltpu.VMEM((_NFLAT,), jnp.float32),
    ],
)(_k4_body)


def _k4b_body(sp_ref, s_ref):
    s_ref[...] = jnp.sum(sp_ref[...], axis=0)


def _combine_sums(sp):
    return pl.pallas_call(
        _k4b_body,
        out_shape=jax.ShapeDtypeStruct((_NFLAT,), jnp.float32),
    )(sp)


_GROUPS = CHUNK * CHANNELS // 16
_CHUNKF = CHUNK * CHANNELS


def _k5_body(ef_hbm, row_hbm, sf_hbm, out_hbm, idx, ebuf, sbuf):
    w = lax.axis_index("c") * NS + lax.axis_index("s")
    iota = lax.iota(jnp.int32, 16)
    rsel = iota >> 3
    csel = iota & 7
    pltpu.sync_copy(sf_hbm, sbuf)
    for k in range(NCHUNK):
        base = pl.multiple_of(w * EDGES_PER_W + k * CHUNK, 8)
        fsl = pl.ds(pl.multiple_of(base * CHANNELS, 8), _CHUNKF)
        pltpu.sync_copy(row_hbm.at[pl.ds(base, CHUNK)], idx)
        pltpu.sync_copy(ef_hbm.at[fsl], ebuf)

        def body(j, carry):
            off = pl.multiple_of(16 * j, 8)
            rows2 = plsc.load_gather(idx, [2 * j + rsel])
            den = plsc.load_gather(sbuf, [rows2 * CHANNELS + csel])
            ebuf[pl.ds(off, 16)] = ebuf[pl.ds(off, 16)] / (den + 1e-16)
            return carry

        lax.fori_loop(0, _GROUPS, body, 0)
        pltpu.sync_copy(ebuf, out_hbm.at[fsl])


_normalize = functools.partial(
    pl.kernel,
    compiler_params=_SC_PARAMS,
    out_type=jax.ShapeDtypeStruct((N_EDGES * CHANNELS,), jnp.float32),
    mesh=_MESH,
    scratch_types=[
        pltpu.VMEM((CHUNK,), jnp.int32),
        pltpu.VMEM((_CHUNKF,), jnp.float32),
        pltpu.VMEM((N_NODES * CHANNELS,), jnp.float32),
    ],
)(_k5_body)


def kernel(x, edge_index, W1, b1, W2, b2):
    row = edge_index[0].astype(jnp.int32)
    col = edge_index[1].astype(jnp.int32)
    t = _node_table(x, W1)
    h = _gather_sum(t, row, col)
    e = _edge_exp_scores(h, b1, W2, b2)
    ef = e.reshape(-1)
    sp = _segment_sums(ef, row)
    s = _combine_sums(sp)
    out_flat = _normalize(ef, row, s)
    return out_flat.reshape(N_EDGES, CHANNELS)


# speedup vs baseline: 5.1542x; 1.1558x over previous
import functools

import jax
import jax.numpy as jnp
from jax import lax
from jax.experimental import pallas as pl
from jax.experimental.pallas import tpu as pltpu
from jax.experimental.pallas import tpu_sc as plsc

N_NODES = 10000
N_EDGES = 320000
NFEATS = 128
HIDDEN = 64
CHANNELS = 8

NC = 2
NS = 16
NW = NC * NS
EDGES_PER_W = N_EDGES // NW
CHUNK = 200
NCHUNK = EDGES_PER_W // CHUNK
ROWS_PER_TILE = 624
ROWS_LAST = N_NODES - 15 * ROWS_PER_TILE

_MESH = plsc.VectorSubcoreMesh(core_axis_name="c", subcore_axis_name="s",
                               num_cores=NC, num_subcores=NS)
_SC_PARAMS = pltpu.CompilerParams(needs_layout_passes=False)


def _k1_body(x_ref, w1_ref, t_ref):
    xv = x_ref[...]
    w = w1_ref[...]
    t_ref[:, :HIDDEN] = jnp.dot(xv, w[:NFEATS], preferred_element_type=jnp.float32)
    t_ref[:, HIDDEN:] = jnp.dot(xv, w[NFEATS:], preferred_element_type=jnp.float32)


def _node_table(x, W1):
    return pl.pallas_call(
        _k1_body,
        out_shape=jax.ShapeDtypeStruct((N_NODES, 2 * HIDDEN), jnp.float32),
    )(x, W1)


def _k2_body(t_hbm, row_hbm, col_hbm, h_hbm, idxr, idxc, bufr, bufc, gbuf, sem):
    w = lax.axis_index("c") * NS + lax.axis_index("s")
    for k in range(NCHUNK):
        base = pl.multiple_of(w * EDGES_PER_W + k * CHUNK, 8)
        sl = pl.ds(base, CHUNK)
        pltpu.sync_copy(row_hbm.at[sl], idxr)
        pltpu.sync_copy(col_hbm.at[sl], idxc)
        d1 = pltpu.async_copy(t_hbm.at[idxr], bufr, sem)
        d2 = pltpu.async_copy(t_hbm.at[idxc], bufc, sem)
        d1.wait()
        d2.wait()

        @plsc.parallel_loop(0, CHUNK, unroll=8)
        def _(e):
            for j in range(HIDDEN // 16):
                a = bufr[e, pl.ds(j * 16, 16)]
                b = bufc[e, pl.ds(HIDDEN + j * 16, 16)]
                gbuf[e, pl.ds(j * 16, 16)] = a + b

        pltpu.sync_copy(gbuf, h_hbm.at[sl])


_gather_sum = functools.partial(
    pl.kernel,
    compiler_params=_SC_PARAMS,
    out_type=jax.ShapeDtypeStruct((N_EDGES, HIDDEN), jnp.float32),
    mesh=_MESH,
    scratch_types=[
        pltpu.VMEM((CHUNK,), jnp.int32),
        pltpu.VMEM((CHUNK,), jnp.int32),
        pltpu.VMEM((CHUNK, 2 * HIDDEN), jnp.float32),
        pltpu.VMEM((CHUNK, 2 * HIDDEN), jnp.float32),
        pltpu.VMEM((CHUNK, HIDDEN), jnp.float32),
        pltpu.SemaphoreType.DMA,
    ],
)(_k2_body)


_K3_BE = 6400


def _k3_body(h_ref, b1_ref, w2_ref, b2_ref, o_ref):
    h = jnp.maximum(h_ref[...] + b1_ref[...][None, :], 0.0)
    s = jnp.dot(h, w2_ref[...], preferred_element_type=jnp.float32)
    o_ref[...] = jnp.exp(s + b2_ref[...][None, :])


def _edge_exp_scores(h, b1, W2, b2):
    grid = (N_EDGES // _K3_BE,)
    return pl.pallas_call(
        _k3_body,
        grid=grid,
        in_specs=[
            pl.BlockSpec((_K3_BE, HIDDEN), lambda i: (i, 0)),
            pl.BlockSpec((HIDDEN,), lambda i: (0,)),
            pl.BlockSpec((HIDDEN, CHANNELS), lambda i: (0, 0)),
            pl.BlockSpec((CHANNELS,), lambda i: (0,)),
        ],
        out_specs=pl.BlockSpec((_K3_BE, CHANNELS), lambda i: (i, 0)),
        out_shape=jax.ShapeDtypeStruct((N_EDGES, CHANNELS), jnp.float32),
    )(h, b1, W2, b2)


_NFLAT = N_NODES * CHANNELS
_CHUNK4 = 400
_NCHUNK4 = EDGES_PER_W // _CHUNK4
_GROUPS4 = _CHUNK4 * CHANNELS // 16
_ZGROUPS = _NFLAT // 16


def _k4_body(ef_hbm, row_hbm, sp_hbm, idx, ebuf, sbuf):
    w = lax.axis_index("c") * NS + lax.axis_index("s")
    iota = lax.iota(jnp.int32, 16)
    rsel = iota >> 3
    csel = iota & 7
    mlo = iota < 8
    mhi = iota >= 8
    zero16 = jnp.zeros((16,), jnp.float32)

    @plsc.parallel_loop(0, _ZGROUPS, unroll=8)
    def _(i):
        sbuf[pl.ds(pl.multiple_of(16 * i, 8), 16)] = zero16

    for k in range(_NCHUNK4):
        base = pl.multiple_of(w * EDGES_PER_W + k * _CHUNK4, 8)
        fsl = pl.ds(pl.multiple_of(base * CHANNELS, 8), _CHUNK4 * CHANNELS)
        pltpu.sync_copy(row_hbm.at[pl.ds(base, _CHUNK4)], idx)
        pltpu.sync_copy(ef_hbm.at[fsl], ebuf)

        @plsc.parallel_loop(0, _GROUPS4, unroll=8)
        def _(j):
            off = pl.multiple_of(16 * j, 8)
            rows2 = plsc.load_gather(idx, [2 * j + rsel])
            eidx = rows2 * CHANNELS + csel
            ev = ebuf[pl.ds(off, 16)]
            plsc.addupdate_scatter(sbuf, [eidx], ev, mask=mlo)
            plsc.addupdate_scatter(sbuf, [eidx], ev, mask=mhi)

    pltpu.sync_copy(sbuf, sp_hbm.at[w])


_segment_sums = functools.partial(
    pl.kernel,
    compiler_params=_SC_PARAMS,
    out_type=jax.ShapeDtypeStruct((NW, _NFLAT), jnp.float32),
    mesh=_MESH,
    scratch_types=[
        pltpu.VMEM((_CHUNK4,), jnp.int32),
        pltpu.VMEM((_CHUNK4 * CHANNELS,), jnp.float32),
        pltpu.VMEM((_NFLAT,), jnp.float32),
    ],
)(_k4_body)


def _k4b_body(sp_ref, s_ref):
    s_ref[...] = jnp.sum(sp_ref[...], axis=0)


def _combine_sums(sp):
    return pl.pallas_call(
        _k4b_body,
        out_shape=jax.ShapeDtypeStruct((_NFLAT,), jnp.float32),
    )(sp)


_GROUPS = CHUNK * CHANNELS // 16
_CHUNKF = CHUNK * CHANNELS


def _k5_body(ef_hbm, row_hbm, sf_hbm, out_hbm, idx, ebuf, sbuf):
    w = lax.axis_index("c") * NS + lax.axis_index("s")
    iota = lax.iota(jnp.int32, 16)
    rsel = iota >> 3
    csel = iota & 7
    pltpu.sync_copy(sf_hbm, sbuf)
    for k in range(NCHUNK):
        base = pl.multiple_of(w * EDGES_PER_W + k * CHUNK, 8)
        fsl = pl.ds(pl.multiple_of(base * CHANNELS, 8), _CHUNKF)
        pltpu.sync_copy(row_hbm.at[pl.ds(base, CHUNK)], idx)
        pltpu.sync_copy(ef_hbm.at[fsl], ebuf)

        @plsc.parallel_loop(0, _GROUPS, unroll=8)
        def _(j):
            off = pl.multiple_of(16 * j, 8)
            rows2 = plsc.load_gather(idx, [2 * j + rsel])
            den = plsc.load_gather(sbuf, [rows2 * CHANNELS + csel])
            ebuf[pl.ds(off, 16)] = ebuf[pl.ds(off, 16)] / (den + 1e-16)
        pltpu.sync_copy(ebuf, out_hbm.at[fsl])


_normalize = functools.partial(
    pl.kernel,
    compiler_params=_SC_PARAMS,
    out_type=jax.ShapeDtypeStruct((N_EDGES * CHANNELS,), jnp.float32),
    mesh=_MESH,
    scratch_types=[
        pltpu.VMEM((CHUNK,), jnp.int32),
        pltpu.VMEM((_CHUNKF,), jnp.float32),
        pltpu.VMEM((N_NODES * CHANNELS,), jnp.float32),
    ],
)(_k5_body)


def kernel(x, edge_index, W1, b1, W2, b2):
    row = edge_index[0].astype(jnp.int32)
    col = edge_index[1].astype(jnp.int32)
    t = _node_table(x, W1)
    h = _gather_sum(t, row, col)
    ef = _edge_exp_scores(h, b1, W2, b2).reshape(-1)
    sp = _segment_sums(ef, row)
    s = _combine_sums(sp)
    out_flat = _normalize(ef, row, s)
    return out_flat.reshape(N_EDGES, CHANNELS)


# speedup vs baseline: 6.0545x; 1.1747x over previous
import functools

import jax
import jax.numpy as jnp
from jax import lax
from jax.experimental import pallas as pl
from jax.experimental.pallas import tpu as pltpu
from jax.experimental.pallas import tpu_sc as plsc

N_NODES = 10000
N_EDGES = 320000
NFEATS = 128
HIDDEN = 64
CHANNELS = 8

NC = 2
NS = 16
NW = NC * NS
EDGES_PER_W = N_EDGES // NW
CHUNK = 200
NCHUNK = EDGES_PER_W // CHUNK
ROWS_PER_TILE = 624
ROWS_LAST = N_NODES - 15 * ROWS_PER_TILE

_MESH = plsc.VectorSubcoreMesh(core_axis_name="c", subcore_axis_name="s",
                               num_cores=NC, num_subcores=NS)
_SC_PARAMS = pltpu.CompilerParams(needs_layout_passes=False)


def _k1_body(x_ref, w1_ref, t_ref):
    xv = x_ref[...]
    w = w1_ref[...]
    t_ref[:, :HIDDEN] = jnp.dot(xv, w[:NFEATS], preferred_element_type=jnp.float32)
    t_ref[:, HIDDEN:] = jnp.dot(xv, w[NFEATS:], preferred_element_type=jnp.float32)


def _node_table(x, W1):
    return pl.pallas_call(
        _k1_body,
        out_shape=jax.ShapeDtypeStruct((N_NODES, 2 * HIDDEN), jnp.float32),
    )(x, W1)


_K2C = 80
_K2N = EDGES_PER_W // _K2C


def _k2_body(t_hbm, row_hbm, col_hbm, h_hbm,
             idxr, idxc, bufr, bufc, gbuf, gsem, osem):
    sid = lax.axis_index("s")
    w = lax.axis_index("c") * NS + sid

    me = pl.ds(pl.multiple_of(w * EDGES_PER_W, 8), EDGES_PER_W)
    pltpu.sync_copy(row_hbm.at[me], idxr)
    pltpu.sync_copy(col_hbm.at[me], idxc)
    plsc.subcore_barrier()

    def issue(k, p):
        isl = pl.ds(pl.multiple_of(k * _K2C, 8), _K2C)
        pltpu.async_copy(t_hbm.at[idxr.at[isl]], bufr[p], gsem)
        pltpu.async_copy(t_hbm.at[idxc.at[isl]], bufc[p], gsem)

    def wait_gather(p):
        pltpu.make_async_copy(t_hbm.at[pl.ds(0, _K2C)], bufr[p], gsem).wait()
        pltpu.make_async_copy(t_hbm.at[pl.ds(0, _K2C)], bufc[p], gsem).wait()

    def wait_out(p):
        pltpu.make_async_copy(gbuf[p], h_hbm.at[pl.ds(0, _K2C)], osem).wait()

    def compute(p):
        @plsc.parallel_loop(0, _K2C, unroll=4)
        def _(e):
            for j in range(HIDDEN // 16):
                a = bufr[p][e, pl.ds(j * 16, 16)]
                b = bufc[p][e, pl.ds(HIDDEN + j * 16, 16)]
                gbuf[p][e, pl.ds(j * 16, 16)] = a + b

    def write_out(k, p):
        base = pl.multiple_of(w * EDGES_PER_W + k * _K2C, 8)
        pltpu.async_copy(gbuf[p], h_hbm.at[pl.ds(base, _K2C)], osem)

    issue(0, 0)
    issue(1, 1)

    def pair(g, carry):
        for p in range(2):
            k = 2 * g + p
            wait_gather(p)

            @pl.when(g > 0)
            def _():
                wait_out(p)

            compute(p)
            write_out(k, p)

            @pl.when(2 * g + p + 2 < _K2N)
            def _():
                issue(2 * g + p + 2, p)

        return carry

    lax.fori_loop(0, _K2N // 2, pair, 0)
    if _K2N % 2 == 1:
        k = _K2N - 1
        wait_gather(0)
        wait_out(0)
        compute(0)
        write_out(k, 0)
    wait_out(0)
    wait_out(1)


_gather_sum = functools.partial(
    pl.kernel,
    compiler_params=_SC_PARAMS,
    out_type=jax.ShapeDtypeStruct((N_EDGES, HIDDEN), jnp.float32),
    mesh=_MESH,
    scratch_types=[
        pltpu.VMEM((EDGES_PER_W,), jnp.int32),
        pltpu.VMEM((EDGES_PER_W,), jnp.int32),
        [pltpu.VMEM((_K2C, 2 * HIDDEN), jnp.float32)] * 2,
        [pltpu.VMEM((_K2C, 2 * HIDDEN), jnp.float32)] * 2,
        [pltpu.VMEM((_K2C, HIDDEN), jnp.float32)] * 2,
        pltpu.SemaphoreType.DMA,
        pltpu.SemaphoreType.DMA,
    ],
)(_k2_body)


_K3_BE = 6400


def _k3_body(h_ref, b1_ref, w2_ref, b2_ref, o_ref):
    h = jnp.maximum(h_ref[...].astype(jnp.float32) + b1_ref[...][None, :], 0.0)
    s = jnp.dot(h, w2_ref[...], preferred_element_type=jnp.float32)
    o_ref[...] = jnp.exp(s + b2_ref[...][None, :])


def _edge_exp_scores(h, b1, W2, b2):
    grid = (N_EDGES // _K3_BE,)
    return pl.pallas_call(
        _k3_body,
        grid=grid,
        in_specs=[
            pl.BlockSpec((_K3_BE, HIDDEN), lambda i: (i, 0)),
            pl.BlockSpec((HIDDEN,), lambda i: (0,)),
            pl.BlockSpec((HIDDEN, CHANNELS), lambda i: (0, 0)),
            pl.BlockSpec((CHANNELS,), lambda i: (0,)),
        ],
        out_specs=pl.BlockSpec((_K3_BE, CHANNELS), lambda i: (i, 0)),
        out_shape=jax.ShapeDtypeStruct((N_EDGES, CHANNELS), jnp.float32),
    )(h, b1, W2, b2)


_NFLAT = N_NODES * CHANNELS
_CHUNK4 = 400
_NCHUNK4 = EDGES_PER_W // _CHUNK4
_GROUPS4 = _CHUNK4 * CHANNELS // 16
_ZGROUPS = _NFLAT // 16


def _k4_body(ef_hbm, row_hbm, sp_hbm, idx, ebuf, sbuf):
    w = lax.axis_index("c") * NS + lax.axis_index("s")
    iota = lax.iota(jnp.int32, 16)
    rsel = iota >> 3
    csel = iota & 7
    mlo = iota < 8
    mhi = iota >= 8
    zero16 = jnp.zeros((16,), jnp.float32)

    @plsc.parallel_loop(0, _ZGROUPS, unroll=8)
    def _(i):
        sbuf[pl.ds(pl.multiple_of(16 * i, 8), 16)] = zero16

    for k in range(_NCHUNK4):
        base = pl.multiple_of(w * EDGES_PER_W + k * _CHUNK4, 8)
        fsl = pl.ds(pl.multiple_of(base * CHANNELS, 8), _CHUNK4 * CHANNELS)
        pltpu.sync_copy(row_hbm.at[pl.ds(base, _CHUNK4)], idx)
        pltpu.sync_copy(ef_hbm.at[fsl], ebuf)

        @plsc.parallel_loop(0, _GROUPS4, unroll=8)
        def _(j):
            off = pl.multiple_of(16 * j, 8)
            rows2 = plsc.load_gather(idx, [2 * j + rsel])
            eidx = rows2 * CHANNELS + csel
            ev = ebuf[pl.ds(off, 16)]
            plsc.addupdate_scatter(sbuf, [eidx], ev, mask=mlo)
            plsc.addupdate_scatter(sbuf, [eidx], ev, mask=mhi)

    pltpu.sync_copy(sbuf, sp_hbm.at[w])


_segment_sums = functools.partial(
    pl.kernel,
    compiler_params=_SC_PARAMS,
    out_type=jax.ShapeDtypeStruct((NW, _NFLAT), jnp.float32),
    mesh=_MESH,
    scratch_types=[
        pltpu.VMEM((_CHUNK4,), jnp.int32),
        pltpu.VMEM((_CHUNK4 * CHANNELS,), jnp.float32),
        pltpu.VMEM((_NFLAT,), jnp.float32),
    ],
)(_k4_body)


def _k4b_body(sp_ref, s_ref):
    s_ref[...] = jnp.sum(sp_ref[...], axis=0)


def _combine_sums(sp):
    return pl.pallas_call(
        _k4b_body,
        out_shape=jax.ShapeDtypeStruct((_NFLAT,), jnp.float32),
    )(sp)


_GROUPS = CHUNK * CHANNELS // 16
_CHUNKF = CHUNK * CHANNELS


def _k5_body(ef_hbm, row_hbm, sf_hbm, out_hbm, idx, ebuf, sbuf):
    w = lax.axis_index("c") * NS + lax.axis_index("s")
    iota = lax.iota(jnp.int32, 16)
    rsel = iota >> 3
    csel = iota & 7
    pltpu.sync_copy(sf_hbm, sbuf)
    for k in range(NCHUNK):
        base = pl.multiple_of(w * EDGES_PER_W + k * CHUNK, 8)
        fsl = pl.ds(pl.multiple_of(base * CHANNELS, 8), _CHUNKF)
        pltpu.sync_copy(row_hbm.at[pl.ds(base, CHUNK)], idx)
        pltpu.sync_copy(ef_hbm.at[fsl], ebuf)

        @plsc.parallel_loop(0, _GROUPS, unroll=8)
        def _(j):
            off = pl.multiple_of(16 * j, 8)
            rows2 = plsc.load_gather(idx, [2 * j + rsel])
            den = plsc.load_gather(sbuf, [rows2 * CHANNELS + csel])
            ebuf[pl.ds(off, 16)] = ebuf[pl.ds(off, 16)] / (den + 1e-16)
        pltpu.sync_copy(ebuf, out_hbm.at[fsl])


_normalize = functools.partial(
    pl.kernel,
    compiler_params=_SC_PARAMS,
    out_type=jax.ShapeDtypeStruct((N_EDGES * CHANNELS,), jnp.float32),
    mesh=_MESH,
    scratch_types=[
        pltpu.VMEM((CHUNK,), jnp.int32),
        pltpu.VMEM((_CHUNKF,), jnp.float32),
        pltpu.VMEM((N_NODES * CHANNELS,), jnp.float32),
    ],
)(_k5_body)


def kernel(x, edge_index, W1, b1, W2, b2):
    row = edge_index[0].astype(jnp.int32)
    col = edge_index[1].astype(jnp.int32)
    t = _node_table(x, W1)
    h = _gather_sum(t, row, col)
    ef = _edge_exp_scores(h, b1, W2, b2).reshape(-1)
    sp = _segment_sums(ef, row)
    s = _combine_sums(sp)
    out_flat = _normalize(ef, row, s)
    return out_flat.reshape(N_EDGES, CHANNELS)


# speedup vs baseline: 6.0745x; 1.0033x over previous
import functools

import jax
import jax.numpy as jnp
from jax import lax
from jax.experimental import pallas as pl
from jax.experimental.pallas import tpu as pltpu
from jax.experimental.pallas import tpu_sc as plsc

N_NODES = 10000
N_EDGES = 320000
NFEATS = 128
HIDDEN = 64
CHANNELS = 8

NC = 2
NS = 16
NW = NC * NS
EDGES_PER_W = N_EDGES // NW
CHUNK = 200
NCHUNK = EDGES_PER_W // CHUNK
ROWS_PER_TILE = 624
ROWS_LAST = N_NODES - 15 * ROWS_PER_TILE

_MESH = plsc.VectorSubcoreMesh(core_axis_name="c", subcore_axis_name="s",
                               num_cores=NC, num_subcores=NS)
_SC_PARAMS = pltpu.CompilerParams(needs_layout_passes=False)


def _k1_body(x_ref, w1_ref, t_ref):
    xv = x_ref[...]
    w = w1_ref[...]
    t_ref[:, :HIDDEN] = jnp.dot(xv, w[:NFEATS], preferred_element_type=jnp.float32)
    t_ref[:, HIDDEN:] = jnp.dot(xv, w[NFEATS:], preferred_element_type=jnp.float32)


def _node_table(x, W1):
    return pl.pallas_call(
        _k1_body,
        out_shape=jax.ShapeDtypeStruct((N_NODES, 2 * HIDDEN), jnp.float32),
    )(x, W1)


_K2C = 80
_K2N = EDGES_PER_W // _K2C


def _k2_body(t_hbm, row_hbm, col_hbm, h_hbm,
             idxr, idxc, bufr, bufc, gbuf, gsem, osem):
    sid = lax.axis_index("s")
    w = lax.axis_index("c") * NS + sid

    me = pl.ds(pl.multiple_of(w * EDGES_PER_W, 8), EDGES_PER_W)
    pltpu.sync_copy(row_hbm.at[me], idxr)
    pltpu.sync_copy(col_hbm.at[me], idxc)
    plsc.subcore_barrier()

    def issue(k, p):
        isl = pl.ds(pl.multiple_of(k * _K2C, 8), _K2C)
        pltpu.async_copy(t_hbm.at[idxr.at[isl]], bufr[p], gsem)
        pltpu.async_copy(t_hbm.at[idxc.at[isl]], bufc[p], gsem)

    def wait_gather(p):
        pltpu.make_async_copy(t_hbm.at[pl.ds(0, _K2C)], bufr[p], gsem).wait()
        pltpu.make_async_copy(t_hbm.at[pl.ds(0, _K2C)], bufc[p], gsem).wait()

    def wait_out(p):
        pltpu.make_async_copy(gbuf[p], h_hbm.at[pl.ds(0, _K2C)], osem).wait()

    def compute(p):
        @plsc.parallel_loop(0, _K2C, unroll=4)
        def _(e):
            for j in range(HIDDEN // 16):
                a = bufr[p][e, pl.ds(j * 16, 16)]
                b = bufc[p][e, pl.ds(HIDDEN + j * 16, 16)]
                gbuf[p][e, pl.ds(j * 16, 16)] = a + b

    def write_out(k, p):
        base = pl.multiple_of(w * EDGES_PER_W + k * _K2C, 8)
        pltpu.async_copy(gbuf[p], h_hbm.at[pl.ds(base, _K2C)], osem)

    issue(0, 0)
    issue(1, 1)

    def pair(g, carry):
        for p in range(2):
            k = 2 * g + p
            wait_gather(p)

            @pl.when(g > 0)
            def _():
                wait_out(p)

            compute(p)
            write_out(k, p)

            @pl.when(2 * g + p + 2 < _K2N)
            def _():
                issue(2 * g + p + 2, p)

        return carry

    lax.fori_loop(0, _K2N // 2, pair, 0)
    if _K2N % 2 == 1:
        k = _K2N - 1
        wait_gather(0)
        wait_out(0)
        compute(0)
        write_out(k, 0)
    wait_out(0)
    wait_out(1)


_gather_sum = functools.partial(
    pl.kernel,
    compiler_params=_SC_PARAMS,
    out_type=jax.ShapeDtypeStruct((N_EDGES, HIDDEN), jnp.float32),
    mesh=_MESH,
    scratch_types=[
        pltpu.VMEM((EDGES_PER_W,), jnp.int32),
        pltpu.VMEM((EDGES_PER_W,), jnp.int32),
        [pltpu.VMEM((_K2C, 2 * HIDDEN), jnp.float32)] * 2,
        [pltpu.VMEM((_K2C, 2 * HIDDEN), jnp.float32)] * 2,
        [pltpu.VMEM((_K2C, HIDDEN), jnp.float32)] * 2,
        pltpu.SemaphoreType.DMA,
        pltpu.SemaphoreType.DMA,
    ],
)(_k2_body)


_K3_BE = 6400


def _k3_body(h_ref, b1_ref, w2_ref, b2_ref, o_ref):
    h = jnp.maximum(h_ref[...].astype(jnp.float32) + b1_ref[...][None, :], 0.0)
    s = jnp.dot(h, w2_ref[...], preferred_element_type=jnp.float32)
    o_ref[...] = jnp.exp(s + b2_ref[...][None, :])


def _edge_exp_scores(h, b1, W2, b2):
    grid = (N_EDGES // _K3_BE,)
    return pl.pallas_call(
        _k3_body,
        grid=grid,
        in_specs=[
            pl.BlockSpec((_K3_BE, HIDDEN), lambda i: (i, 0)),
            pl.BlockSpec((HIDDEN,), lambda i: (0,)),
            pl.BlockSpec((HIDDEN, CHANNELS), lambda i: (0, 0)),
            pl.BlockSpec((CHANNELS,), lambda i: (0,)),
        ],
        out_specs=pl.BlockSpec((_K3_BE, CHANNELS), lambda i: (i, 0)),
        out_shape=jax.ShapeDtypeStruct((N_EDGES, CHANNELS), jnp.float32),
    )(h, b1, W2, b2)


_NFLAT = N_NODES * CHANNELS
_CHUNK4 = 400
_NCHUNK4 = EDGES_PER_W // _CHUNK4
_GROUPS4 = _CHUNK4 * CHANNELS // 16
_ZGROUPS = _NFLAT // 16


def _k4_body(ef_hbm, row_hbm, sp_hbm, idx, ebuf, sbuf):
    w = lax.axis_index("c") * NS + lax.axis_index("s")
    iota = lax.iota(jnp.int32, 16)
    rsel = iota >> 3
    csel = iota & 7
    mlo = iota < 8
    mhi = iota >= 8
    zero16 = jnp.zeros((16,), jnp.float32)

    @plsc.parallel_loop(0, _ZGROUPS, unroll=8)
    def _(i):
        sbuf[pl.ds(pl.multiple_of(16 * i, 8), 16)] = zero16

    for k in range(_NCHUNK4):
        base = pl.multiple_of(w * EDGES_PER_W + k * _CHUNK4, 8)
        fsl = pl.ds(pl.multiple_of(base * CHANNELS, 8), _CHUNK4 * CHANNELS)
        pltpu.sync_copy(row_hbm.at[pl.ds(base, _CHUNK4)], idx)
        pltpu.sync_copy(ef_hbm.at[fsl], ebuf)

        @plsc.parallel_loop(0, _GROUPS4, unroll=8)
        def _(j):
            off = pl.multiple_of(16 * j, 8)
            rows2 = plsc.load_gather(idx, [2 * j + rsel])
            eidx = rows2 * CHANNELS + csel
            ev = ebuf[pl.ds(off, 16)]
            plsc.addupdate_scatter(sbuf, [eidx], ev, mask=mlo)
            plsc.addupdate_scatter(sbuf, [eidx], ev, mask=mhi)

    pltpu.sync_copy(sbuf, sp_hbm.at[w])


_segment_sums = functools.partial(
    pl.kernel,
    compiler_params=_SC_PARAMS,
    out_type=jax.ShapeDtypeStruct((NW, _NFLAT), jnp.float32),
    mesh=_MESH,
    scratch_types=[
        pltpu.VMEM((_CHUNK4,), jnp.int32),
        pltpu.VMEM((_CHUNK4 * CHANNELS,), jnp.float32),
        pltpu.VMEM((_NFLAT,), jnp.float32),
    ],
)(_k4_body)


def _k4b_body(sp_ref, s_ref):
    s_ref[...] = 1.0 / (jnp.sum(sp_ref[...], axis=0) + 1e-16)


def _combine_sums(sp):
    return pl.pallas_call(
        _k4b_body,
        out_shape=jax.ShapeDtypeStruct((_NFLAT,), jnp.float32),
    )(sp)


_GROUPS = CHUNK * CHANNELS // 16
_CHUNKF = CHUNK * CHANNELS


def _k5_body(ef_hbm, row_hbm, sf_hbm, out_hbm, idx, ebuf, sbuf):
    w = lax.axis_index("c") * NS + lax.axis_index("s")
    iota = lax.iota(jnp.int32, 16)
    rsel = iota >> 3
    csel = iota & 7
    pltpu.sync_copy(sf_hbm, sbuf)
    for k in range(NCHUNK):
        base = pl.multiple_of(w * EDGES_PER_W + k * CHUNK, 8)
        fsl = pl.ds(pl.multiple_of(base * CHANNELS, 8), _CHUNKF)
        pltpu.sync_copy(row_hbm.at[pl.ds(base, CHUNK)], idx)
        pltpu.sync_copy(ef_hbm.at[fsl], ebuf)

        @plsc.parallel_loop(0, _GROUPS, unroll=8)
        def _(j):
            off = pl.multiple_of(16 * j, 8)
            rows2 = plsc.load_gather(idx, [2 * j + rsel])
            rcp = plsc.load_gather(sbuf, [rows2 * CHANNELS + csel])
            ebuf[pl.ds(off, 16)] = ebuf[pl.ds(off, 16)] * rcp
        pltpu.sync_copy(ebuf, out_hbm.at[fsl])


_normalize = functools.partial(
    pl.kernel,
    compiler_params=_SC_PARAMS,
    out_type=jax.ShapeDtypeStruct((N_EDGES * CHANNELS,), jnp.float32),
    mesh=_MESH,
    scratch_types=[
        pltpu.VMEM((CHUNK,), jnp.int32),
        pltpu.VMEM((_CHUNKF,), jnp.float32),
        pltpu.VMEM((N_NODES * CHANNELS,), jnp.float32),
    ],
)(_k5_body)


def kernel(x, edge_index, W1, b1, W2, b2):
    row = edge_index[0].astype(jnp.int32)
    col = edge_index[1].astype(jnp.int32)
    t = _node_table(x, W1)
    h = _gather_sum(t, row, col)
    ef = _edge_exp_scores(h, b1, W2, b2).reshape(-1)
    sp = _segment_sums(ef, row)
    s = _combine_sums(sp)
    out_flat = _normalize(ef, row, s)
    return out_flat.reshape(N_EDGES, CHANNELS)


# speedup vs baseline: 6.8731x; 1.1315x over previous
import functools

import jax
import jax.numpy as jnp
from jax import lax
from jax.experimental import pallas as pl
from jax.experimental.pallas import tpu as pltpu
from jax.experimental.pallas import tpu_sc as plsc

N_NODES = 10000
N_EDGES = 320000
NFEATS = 128
HIDDEN = 64
CHANNELS = 8

NC = 2
NS = 16
NW = NC * NS
EDGES_PER_W = N_EDGES // NW
CHUNK = 200
NCHUNK = EDGES_PER_W // CHUNK
ROWS_PER_TILE = 624
ROWS_LAST = N_NODES - 15 * ROWS_PER_TILE

_MESH = plsc.VectorSubcoreMesh(core_axis_name="c", subcore_axis_name="s",
                               num_cores=NC, num_subcores=NS)
_SC_PARAMS = pltpu.CompilerParams(needs_layout_passes=False)


def _k1_body(x_ref, w1_ref, t_ref):
    xv = x_ref[...]
    w = w1_ref[...]
    t_ref[:, :HIDDEN] = jnp.dot(xv, w[:NFEATS], preferred_element_type=jnp.float32)
    t_ref[:, HIDDEN:] = jnp.dot(xv, w[NFEATS:], preferred_element_type=jnp.float32)


def _node_table(x, W1):
    return pl.pallas_call(
        _k1_body,
        out_shape=jax.ShapeDtypeStruct((N_NODES, 2 * HIDDEN), jnp.float32),
    )(x, W1)


_K2C = 80
_K2N = EDGES_PER_W // _K2C


def _k2_body(t_hbm, row_hbm, col_hbm, h_hbm,
             idxr, idxc, bufr, bufc, gbuf, gsem, osem):
    sid = lax.axis_index("s")
    w = lax.axis_index("c") * NS + sid

    me = pl.ds(pl.multiple_of(w * EDGES_PER_W, 8), EDGES_PER_W)
    pltpu.sync_copy(row_hbm.at[me], idxr)
    pltpu.sync_copy(col_hbm.at[me], idxc)
    plsc.subcore_barrier()

    def issue(k, p):
        isl = pl.ds(pl.multiple_of(k * _K2C, 8), _K2C)
        pltpu.async_copy(t_hbm.at[idxr.at[isl]], bufr[p], gsem)
        pltpu.async_copy(t_hbm.at[idxc.at[isl]], bufc[p], gsem)

    def wait_gather(p):
        pltpu.make_async_copy(t_hbm.at[pl.ds(0, _K2C)], bufr[p], gsem).wait()
        pltpu.make_async_copy(t_hbm.at[pl.ds(0, _K2C)], bufc[p], gsem).wait()

    def wait_out(p):
        pltpu.make_async_copy(gbuf[p], h_hbm.at[pl.ds(0, _K2C)], osem).wait()

    def compute(p):
        @plsc.parallel_loop(0, _K2C, unroll=4)
        def _(e):
            for j in range(HIDDEN // 16):
                a = bufr[p][e, pl.ds(j * 16, 16)]
                b = bufc[p][e, pl.ds(HIDDEN + j * 16, 16)]
                gbuf[p][e, pl.ds(j * 16, 16)] = a + b

    def write_out(k, p):
        base = pl.multiple_of(w * EDGES_PER_W + k * _K2C, 8)
        pltpu.async_copy(gbuf[p], h_hbm.at[pl.ds(base, _K2C)], osem)

    issue(0, 0)
    issue(1, 1)

    def pair(g, carry):
        for p in range(2):
            k = 2 * g + p
            wait_gather(p)

            @pl.when(g > 0)
            def _():
                wait_out(p)

            compute(p)
            write_out(k, p)

            @pl.when(2 * g + p + 2 < _K2N)
            def _():
                issue(2 * g + p + 2, p)

        return carry

    lax.fori_loop(0, _K2N // 2, pair, 0)
    if _K2N % 2 == 1:
        k = _K2N - 1
        wait_gather(0)
        wait_out(0)
        compute(0)
        write_out(k, 0)
    wait_out(0)
    wait_out(1)


_gather_sum = functools.partial(
    pl.kernel,
    compiler_params=_SC_PARAMS,
    out_type=jax.ShapeDtypeStruct((N_EDGES, HIDDEN), jnp.float32),
    mesh=_MESH,
    scratch_types=[
        pltpu.VMEM((EDGES_PER_W,), jnp.int32),
        pltpu.VMEM((EDGES_PER_W,), jnp.int32),
        [pltpu.VMEM((_K2C, 2 * HIDDEN), jnp.float32)] * 2,
        [pltpu.VMEM((_K2C, 2 * HIDDEN), jnp.float32)] * 2,
        [pltpu.VMEM((_K2C, HIDDEN), jnp.float32)] * 2,
        pltpu.SemaphoreType.DMA,
        pltpu.SemaphoreType.DMA,
    ],
)(_k2_body)


_K3_BE = 6400


def _k3_body(h_ref, b1_ref, w2_ref, b2_ref, o_ref):
    h = jnp.maximum(h_ref[...].astype(jnp.float32) + b1_ref[...][None, :], 0.0)
    s = jnp.dot(h, w2_ref[...], preferred_element_type=jnp.float32)
    o_ref[...] = jnp.exp(s + b2_ref[...][None, :])


def _edge_exp_scores(h, b1, W2, b2):
    grid = (N_EDGES // _K3_BE,)
    return pl.pallas_call(
        _k3_body,
        grid=grid,
        in_specs=[
            pl.BlockSpec((_K3_BE, HIDDEN), lambda i: (i, 0)),
            pl.BlockSpec((HIDDEN,), lambda i: (0,)),
            pl.BlockSpec((HIDDEN, CHANNELS), lambda i: (0, 0)),
            pl.BlockSpec((CHANNELS,), lambda i: (0,)),
        ],
        out_specs=pl.BlockSpec((_K3_BE, CHANNELS), lambda i: (i, 0)),
        out_shape=jax.ShapeDtypeStruct((N_EDGES, CHANNELS), jnp.float32),
    )(h, b1, W2, b2)


_NFLAT = N_NODES * CHANNELS
_CHUNK4 = 400
_NCHUNK4 = EDGES_PER_W // _CHUNK4
_GROUPS4 = _CHUNK4 * CHANNELS // 16
_ZGROUPS = _NFLAT // 16


def _k4_body(ef_hbm, row_hbm, sp_hbm, idxall, ebuf, sbuf, isem):
    w = lax.axis_index("c") * NS + lax.axis_index("s")
    iota = lax.iota(jnp.int32, 16)
    rsel = iota >> 3
    csel = iota & 7
    mlo = iota < 8
    mhi = iota >= 8
    zero16 = jnp.zeros((16,), jnp.float32)

    me = pl.ds(pl.multiple_of(w * EDGES_PER_W, 8), EDGES_PER_W)
    pltpu.sync_copy(row_hbm.at[me], idxall)

    def issue(k, p):
        base = pl.multiple_of(w * EDGES_PER_W + k * _CHUNK4, 8)
        fsl = pl.ds(pl.multiple_of(base * CHANNELS, 8), _CHUNK4 * CHANNELS)
        pltpu.async_copy(ef_hbm.at[fsl], ebuf[p], isem)

    def wait_in(p):
        pltpu.make_async_copy(
            ef_hbm.at[pl.ds(0, _CHUNK4 * CHANNELS)], ebuf[p], isem).wait()

    @plsc.parallel_loop(0, _ZGROUPS, unroll=8)
    def _(i):
        sbuf[pl.ds(pl.multiple_of(16 * i, 8), 16)] = zero16

    issue(0, 0)
    issue(1, 1)
    for k in range(_NCHUNK4):
        p = k % 2
        wait_in(p)

        @plsc.parallel_loop(0, _GROUPS4, unroll=8)
        def _(j):
            off = pl.multiple_of(16 * j, 8)
            rows2 = plsc.load_gather(idxall, [k * _CHUNK4 + 2 * j + rsel])
            eidx = rows2 * CHANNELS + csel
            ev = ebuf[p][pl.ds(off, 16)]
            plsc.addupdate_scatter(sbuf, [eidx], ev, mask=mlo)
            plsc.addupdate_scatter(sbuf, [eidx], ev, mask=mhi)

        if k + 2 < _NCHUNK4:
            issue(k + 2, p)

    pltpu.sync_copy(sbuf, sp_hbm.at[w])


_segment_sums = functools.partial(
    pl.kernel,
    compiler_params=_SC_PARAMS,
    out_type=jax.ShapeDtypeStruct((NW, _NFLAT), jnp.float32),
    mesh=_MESH,
    scratch_types=[
        pltpu.VMEM((EDGES_PER_W,), jnp.int32),
        [pltpu.VMEM((_CHUNK4 * CHANNELS,), jnp.float32)] * 2,
        pltpu.VMEM((_NFLAT,), jnp.float32),
        pltpu.SemaphoreType.DMA,
    ],
)(_k4_body)


def _k4b_body(sp_ref, s_ref):
    s_ref[...] = 1.0 / (jnp.sum(sp_ref[...], axis=0) + 1e-16)


def _combine_sums(sp):
    return pl.pallas_call(
        _k4b_body,
        out_shape=jax.ShapeDtypeStruct((_NFLAT,), jnp.float32),
    )(sp)


_GROUPS = CHUNK * CHANNELS // 16
_CHUNKF = CHUNK * CHANNELS


def _k5_body(ef_hbm, row_hbm, sf_hbm, out_hbm, idxall, ebuf, obuf, sbuf, isem, osem):
    w = lax.axis_index("c") * NS + lax.axis_index("s")
    iota = lax.iota(jnp.int32, 16)
    rsel = iota >> 3
    csel = iota & 7

    me = pl.ds(pl.multiple_of(w * EDGES_PER_W, 8), EDGES_PER_W)
    pltpu.sync_copy(row_hbm.at[me], idxall)
    pltpu.sync_copy(sf_hbm, sbuf)

    def fslice(k):
        base = pl.multiple_of(w * EDGES_PER_W + k * _CHUNK4, 8)
        return pl.ds(pl.multiple_of(base * CHANNELS, 8), _CHUNK4 * CHANNELS)

    def wait_in(p):
        pltpu.make_async_copy(
            ef_hbm.at[pl.ds(0, _CHUNK4 * CHANNELS)], ebuf[p], isem).wait()

    def wait_out(p):
        pltpu.make_async_copy(
            obuf[p], out_hbm.at[pl.ds(0, _CHUNK4 * CHANNELS)], osem).wait()

    pltpu.async_copy(ef_hbm.at[fslice(0)], ebuf[0], isem)
    pltpu.async_copy(ef_hbm.at[fslice(1)], ebuf[1], isem)
    for k in range(_NCHUNK4):
        p = k % 2
        wait_in(p)
        if k >= 2:
            wait_out(p)

        @plsc.parallel_loop(0, _GROUPS4, unroll=8)
        def _(j):
            off = pl.multiple_of(16 * j, 8)
            rows2 = plsc.load_gather(idxall, [k * _CHUNK4 + 2 * j + rsel])
            rcp = plsc.load_gather(sbuf, [rows2 * CHANNELS + csel])
            obuf[p][pl.ds(off, 16)] = ebuf[p][pl.ds(off, 16)] * rcp

        pltpu.async_copy(obuf[p], out_hbm.at[fslice(k)], osem)
        if k + 2 < _NCHUNK4:
            pltpu.async_copy(ef_hbm.at[fslice(k + 2)], ebuf[p], isem)
    wait_out(0)
    wait_out(1)


_normalize = functools.partial(
    pl.kernel,
    compiler_params=_SC_PARAMS,
    out_type=jax.ShapeDtypeStruct((N_EDGES * CHANNELS,), jnp.float32),
    mesh=_MESH,
    scratch_types=[
        pltpu.VMEM((EDGES_PER_W,), jnp.int32),
        [pltpu.VMEM((_CHUNK4 * CHANNELS,), jnp.float32)] * 2,
        [pltpu.VMEM((_CHUNK4 * CHANNELS,), jnp.float32)] * 2,
        pltpu.VMEM((N_NODES * CHANNELS,), jnp.float32),
        pltpu.SemaphoreType.DMA,
        pltpu.SemaphoreType.DMA,
    ],
)(_k5_body)


def kernel(x, edge_index, W1, b1, W2, b2):
    row = edge_index[0].astype(jnp.int32)
    col = edge_index[1].astype(jnp.int32)
    t = _node_table(x, W1)
    h = _gather_sum(t, row, col)
    ef = _edge_exp_scores(h, b1, W2, b2).reshape(-1)
    sp = _segment_sums(ef, row)
    s = _combine_sums(sp)
    out_flat = _normalize(ef, row, s)
    return out_flat.reshape(N_EDGES, CHANNELS)
